# baseline (speedup 1.0000x reference)
import functools

import jax
import jax.numpy as jnp
from jax import lax
from jax.experimental import pallas as pl
from jax.experimental.pallas import tpu as pltpu
from jax.experimental.pallas import tpu_sc as plsc

_BATCH = 4
_SEQ = 8192
_DIM = 1024
_NUM_WORKERS = 32
_ROWS_PER_WORKER = _SEQ // _NUM_WORKERS
_CHUNK = 32
_NCHUNKS = _ROWS_PER_WORKER // _CHUNK


def _broadcast_table(pos_embedding):
    mesh = plsc.VectorSubcoreMesh(core_axis_name="c", subcore_axis_name="s")

    @functools.partial(
        pl.kernel,
        mesh=mesh,
        out_type=jax.ShapeDtypeStruct((_BATCH, _SEQ, _DIM), jnp.float32),
        scratch_types=[
            pltpu.VMEM((_CHUNK, _DIM), jnp.float32),
            pltpu.VMEM((_CHUNK, _DIM), jnp.float32),
            pltpu.SemaphoreType.DMA,
            pltpu.SemaphoreType.DMA,
            pltpu.SemaphoreType.DMA,
            pltpu.SemaphoreType.DMA,
        ],
    )
    def k(table_hbm, out_hbm, buf0, buf1, rsem0, rsem1, wsem0, wsem1):
        wid = lax.axis_index("s") * 2 + lax.axis_index("c")
        base = wid * _ROWS_PER_WORKER
        bufs = (buf0, buf1)
        rsems = (rsem0, rsem1)
        wsems = (wsem0, wsem1)

        reads = [None] * _NCHUNKS
        writes = [[] for _ in range(_NCHUNKS)]
        reads[0] = pltpu.async_copy(
            table_hbm.at[pl.ds(base, _CHUNK)], bufs[0], rsems[0])
        for i in range(_NCHUNKS):
            p = i % 2
            reads[i].wait()
            if i >= 1:
                for w in writes[i - 1]:
                    w.wait()
            if i + 1 < _NCHUNKS:
                row1 = base + (i + 1) * _CHUNK
                reads[i + 1] = pltpu.async_copy(
                    table_hbm.at[pl.ds(row1, _CHUNK)], bufs[1 - p], rsems[1 - p])
            row0 = base + i * _CHUNK
            for b in range(_BATCH):
                writes[i].append(pltpu.async_copy(
                    bufs[p], out_hbm.at[b, pl.ds(row0, _CHUNK)], wsems[p]))
        for w in writes[_NCHUNKS - 1]:
            w.wait()

    return k(pos_embedding)


def kernel(input_ids, pos_embedding):
    del input_ids
    return _broadcast_table(pos_embedding)


# speedup vs baseline: 1.3424x; 1.3424x over previous
import jax
import jax.numpy as jnp
from jax.experimental import pallas as pl

_BATCH = 4
_SEQ = 8192
_DIM = 1024
_BS = 256


def _tc_body(in_ref, out_ref):
    row = in_ref[...]
    out_ref[...] = jnp.broadcast_to(row[None], (_BATCH, _BS, _DIM))


def kernel(input_ids, pos_embedding):
    del input_ids
    return pl.pallas_call(
        _tc_body,
        grid=(_SEQ // _BS,),
        in_specs=[pl.BlockSpec((_BS, _DIM), lambda i: (i, 0))],
        out_specs=pl.BlockSpec((_BATCH, _BS, _DIM), lambda i: (0, i, 0)),
        out_shape=jax.ShapeDtypeStruct((_BATCH, _SEQ, _DIM), jnp.float32),
    )(pos_embedding)
